# baseline (device time: 106054 ns/iter reference)
import functools

import jax
import jax.numpy as jnp
from jax import lax
from jax.experimental import pallas as pl
from jax.experimental.pallas import tpu as pltpu

N_DEV = 4
SQ = 1024
SQ_SH = SQ // N_DEV
D_MODEL = 1024
H_SH = 8
DH = 128
SKV_USED = 1024
BLK = 64
SCALE = 0.08838834764831843


def _body(x_ref, wq_ref, k_hbm, v_hbm, wo_ref, out_ref,
          xfull, kbuf, vbuf, partial, rsbuf, ctx_s, den_s,
          ag_send, ag_recv, rs_send, rs_recv, kv_sems):
    rank = lax.axis_index("i")
    left = (rank + N_DEV - 1) % N_DEV
    right = (rank + 1) % N_DEV

    h0 = rank * H_SH
    kv_copies = []
    for h in range(H_SH):
        kc = pltpu.make_async_copy(
            k_hbm.at[pl.ds(0, SKV_USED), h0 + h, :], kbuf.at[h],
            kv_sems.at[h])
        vc = pltpu.make_async_copy(
            v_hbm.at[pl.ds(0, SKV_USED), h0 + h, :], vbuf.at[h],
            kv_sems.at[H_SH + h])
        kc.start()
        vc.start()
        kv_copies += [kc, vc]

    barrier_sem = pltpu.get_barrier_semaphore()
    for nbr in (left, right):
        pl.semaphore_signal(barrier_sem, inc=1, device_id=(nbr,),
                            device_id_type=pl.DeviceIdType.MESH)
    pl.semaphore_wait(barrier_sem, 2)

    xfull[pl.ds(rank * SQ_SH, SQ_SH), :] = x_ref[...]

    def ag_hop(h):
        c = (rank + N_DEV - h) % N_DEV
        d = pltpu.make_async_remote_copy(
            src_ref=xfull.at[pl.ds(c * SQ_SH, SQ_SH), :],
            dst_ref=xfull.at[pl.ds(c * SQ_SH, SQ_SH), :],
            send_sem=ag_send.at[h],
            recv_sem=ag_recv.at[h],
            device_id=(right,),
            device_id_type=pl.DeviceIdType.MESH,
        )
        d.start()
        return d

    def rs_step(s, c):
        d = pltpu.make_async_remote_copy(
            src_ref=partial.at[pl.ds(c * SQ_SH, SQ_SH), :],
            dst_ref=rsbuf.at[s],
            send_sem=rs_send.at[s],
            recv_sem=rs_recv.at[s],
            device_id=(right,),
            device_id_type=pl.DeviceIdType.MESH,
        )
        d.start()
        return d

    tri01 = jnp.asarray(
        lax.broadcasted_iota(jnp.int32, (SQ_SH, SQ_SH), 1) // BLK
        <= lax.broadcasted_iota(jnp.int32, (SQ_SH, SQ_SH), 0) // BLK,
        jnp.float32)

    def contrib(c):
        xc = xfull[pl.ds(c * SQ_SH, SQ_SH), :]

        def hbody(h, acc):
            qh = jnp.dot(xc, wq_ref[h],
                         preferred_element_type=jnp.float32) * SCALE
            ctx_s[...] = jnp.zeros((SQ_SH, DH), jnp.float32)
            den_s[...] = jnp.zeros((SQ_SH, 1), jnp.float32)

            def kc_block(kc, masked):
                kh = kbuf[h, pl.ds(kc * SQ_SH, SQ_SH), :]
                vh = vbuf[h, pl.ds(kc * SQ_SH, SQ_SH), :]
                s = lax.dot_general(qh, kh, (((1,), (1,)), ((), ())),
                                    preferred_element_type=jnp.float32)
                e = jnp.exp(s)
                if masked:
                    e = e * tri01
                den_s[...] += jnp.sum(e, axis=1, keepdims=True)
                ctx_s[...] += lax.dot_general(
                    e, vh, (((1,), (0,)), ((), ())),
                    preferred_element_type=jnp.float32)

            for kc in range(N_DEV):
                pl.when(kc < c)(lambda kc=kc: kc_block(kc, False))
                pl.when(kc == c)(lambda kc=kc: kc_block(kc, True))

            ctx = ctx_s[...] / den_s[...]
            return acc + jnp.dot(ctx, wo_ref[h],
                                 preferred_element_type=jnp.float32)

        acc = lax.fori_loop(
            0, H_SH, hbody, jnp.zeros((SQ_SH, D_MODEL), jnp.float32))
        partial[pl.ds(c * SQ_SH, SQ_SH), :] = acc

    d0 = ag_hop(0)
    for cp in kv_copies:
        cp.wait()
    contrib(rank)

    c1 = (rank + N_DEV - 1) % N_DEV
    d0.wait_recv()
    d1 = ag_hop(1)
    contrib(c1)
    rs0 = rs_step(0, c1)

    c2 = (rank + N_DEV - 2) % N_DEV
    d1.wait_recv()
    d2 = ag_hop(2)
    contrib(c2)
    rs0.wait_recv()
    partial[pl.ds(c2 * SQ_SH, SQ_SH), :] += rsbuf[0]
    rs1 = rs_step(1, c2)

    c3 = (rank + 1) % N_DEV
    d2.wait_recv()
    contrib(c3)
    rs1.wait_recv()
    partial[pl.ds(c3 * SQ_SH, SQ_SH), :] += rsbuf[1]
    rs2 = rs_step(2, c3)

    rs2.wait_recv()
    out_ref[...] = partial[pl.ds(rank * SQ_SH, SQ_SH), :] + rsbuf[2]

    for d in (d0, d1, d2, rs0, rs1, rs2):
        d.wait_send()

    @functools.partial(pl.run_scoped, exit_sem=pltpu.SemaphoreType.REGULAR)
    def _(exit_sem):
        for nbr in (left, right):
            pl.semaphore_signal(exit_sem, inc=1, device_id=(nbr,),
                                device_id_type=pl.DeviceIdType.MESH)
        pl.semaphore_wait(exit_sem, 2)


def kernel(x, Wq, K_ext, V_ext, Wo):
    x2 = x.reshape(SQ_SH, D_MODEL)
    k3 = K_ext.reshape(K_ext.shape[1], K_ext.shape[2], DH)
    v3 = V_ext.reshape(V_ext.shape[1], V_ext.shape[2], DH)
    wq3 = Wq.reshape(D_MODEL, H_SH, DH).transpose(1, 0, 2)
    wo3 = Wo.reshape(H_SH, DH, D_MODEL)

    out = pl.pallas_call(
        _body,
        out_shape=jax.ShapeDtypeStruct((SQ_SH, D_MODEL), jnp.float32),
        in_specs=[
            pl.BlockSpec(memory_space=pltpu.VMEM),
            pl.BlockSpec(memory_space=pltpu.VMEM),
            pl.BlockSpec(memory_space=pl.ANY),
            pl.BlockSpec(memory_space=pl.ANY),
            pl.BlockSpec(memory_space=pltpu.VMEM),
        ],
        out_specs=pl.BlockSpec(memory_space=pltpu.VMEM),
        scratch_shapes=[
            pltpu.VMEM((SQ, D_MODEL), jnp.float32),
            pltpu.VMEM((H_SH, SKV_USED, DH), jnp.float32),
            pltpu.VMEM((H_SH, SKV_USED, DH), jnp.float32),
            pltpu.VMEM((SQ, D_MODEL), jnp.float32),
            pltpu.VMEM((N_DEV - 1, SQ_SH, D_MODEL), jnp.float32),
            pltpu.VMEM((SQ_SH, DH), jnp.float32),
            pltpu.VMEM((SQ_SH, 1), jnp.float32),
            pltpu.SemaphoreType.DMA((N_DEV - 1,)),
            pltpu.SemaphoreType.DMA((N_DEV - 1,)),
            pltpu.SemaphoreType.DMA((N_DEV - 1,)),
            pltpu.SemaphoreType.DMA((N_DEV - 1,)),
            pltpu.SemaphoreType.DMA((2 * H_SH,)),
        ],
        compiler_params=pltpu.CompilerParams(
            collective_id=0,
            vmem_limit_bytes=100 * 1024 * 1024,
        ),
    )(x2, wq3, k3, v3, wo3)

    return out.reshape(1, SQ_SH, D_MODEL)


# device time: 92529 ns/iter; 1.1462x vs baseline; 1.1462x over previous
import functools

import jax
import jax.numpy as jnp
from jax import lax
from jax.experimental import pallas as pl
from jax.experimental.pallas import tpu as pltpu

N_DEV = 4
SQ = 1024
SQ_SH = SQ // N_DEV
D_MODEL = 1024
H_SH = 8
DH = 128
SKV_USED = 1024
BLK = 64
SCALE = 0.08838834764831843


def _body(x_ref, wq_ref, k_hbm, v_hbm, wo_ref, out_ref,
          xfull, kbuf, vbuf, partial, rsbuf, ctx_s, den_s,
          ag_send, ag_recv, rs_send, rs_recv, kv_sems):
    rank = lax.axis_index("i")
    left = (rank + N_DEV - 1) % N_DEV
    right = (rank + 1) % N_DEV

    kcopy = pltpu.make_async_copy(k_hbm, kbuf, kv_sems.at[0])
    vcopy = pltpu.make_async_copy(v_hbm, vbuf, kv_sems.at[1])
    kcopy.start()
    vcopy.start()

    barrier_sem = pltpu.get_barrier_semaphore()
    for nbr in (left, right):
        pl.semaphore_signal(barrier_sem, inc=1, device_id=(nbr,),
                            device_id_type=pl.DeviceIdType.MESH)
    pl.semaphore_wait(barrier_sem, 2)

    xfull[pl.ds(rank * SQ_SH, SQ_SH), :] = x_ref[...]

    def ag_hop(h):
        c = (rank + N_DEV - h) % N_DEV
        d = pltpu.make_async_remote_copy(
            src_ref=xfull.at[pl.ds(c * SQ_SH, SQ_SH), :],
            dst_ref=xfull.at[pl.ds(c * SQ_SH, SQ_SH), :],
            send_sem=ag_send.at[h],
            recv_sem=ag_recv.at[h],
            device_id=(right,),
            device_id_type=pl.DeviceIdType.MESH,
        )
        d.start()
        return d

    def rs_step(s, c):
        d = pltpu.make_async_remote_copy(
            src_ref=partial.at[pl.ds(c * SQ_SH, SQ_SH), :],
            dst_ref=rsbuf.at[s],
            send_sem=rs_send.at[s],
            recv_sem=rs_recv.at[s],
            device_id=(right,),
            device_id_type=pl.DeviceIdType.MESH,
        )
        d.start()
        return d

    tri01 = jnp.asarray(
        lax.broadcasted_iota(jnp.int32, (SQ_SH, SQ_SH), 1) // BLK
        <= lax.broadcasted_iota(jnp.int32, (SQ_SH, SQ_SH), 0) // BLK,
        jnp.float32)

    def contrib(c):
        xc = xfull[pl.ds(c * SQ_SH, SQ_SH), :]

        def hbody(h, acc):
            qh = jnp.dot(xc, wq_ref[h],
                         preferred_element_type=jnp.float32) * SCALE
            qh = qh.astype(jnp.bfloat16)
            ctx_s[...] = jnp.zeros((SQ_SH, DH), jnp.float32)
            den_s[...] = jnp.zeros((SQ_SH, 1), jnp.float32)

            def kc_block(kc, masked):
                kh = kbuf[h, pl.ds(kc * SQ_SH, SQ_SH), :]
                vh = vbuf[h, pl.ds(kc * SQ_SH, SQ_SH), :]
                s = lax.dot_general(qh, kh, (((1,), (1,)), ((), ())),
                                    preferred_element_type=jnp.float32)
                e = jnp.exp(s)
                if masked:
                    e = e * tri01
                den_s[...] += jnp.sum(e, axis=1, keepdims=True)
                ctx_s[...] += lax.dot_general(
                    e.astype(jnp.bfloat16), vh, (((1,), (0,)), ((), ())),
                    preferred_element_type=jnp.float32)

            for kc in range(N_DEV):
                pl.when(kc < c)(lambda kc=kc: kc_block(kc, False))
                pl.when(kc == c)(lambda kc=kc: kc_block(kc, True))

            ctx = (ctx_s[...] / den_s[...]).astype(jnp.bfloat16)
            return acc + jnp.dot(ctx, wo_ref[h],
                                 preferred_element_type=jnp.float32)

        acc = lax.fori_loop(
            0, H_SH, hbody, jnp.zeros((SQ_SH, D_MODEL), jnp.float32))
        partial[pl.ds(c * SQ_SH, SQ_SH), :] = acc.astype(jnp.bfloat16)

    d0 = ag_hop(0)
    kcopy.wait()
    vcopy.wait()
    d0.wait_recv()
    d1 = ag_hop(1)
    c1 = (rank + N_DEV - 1) % N_DEV
    contrib(c1)
    rs0 = rs_step(0, c1)

    d1.wait_recv()
    d2 = ag_hop(2)
    c2 = (rank + N_DEV - 2) % N_DEV
    contrib(c2)
    rs0.wait_recv()
    partial[pl.ds(c2 * SQ_SH, SQ_SH), :] += rsbuf[0]
    rs1 = rs_step(1, c2)

    d2.wait_recv()
    c3 = (rank + 1) % N_DEV
    contrib(c3)
    rs1.wait_recv()
    partial[pl.ds(c3 * SQ_SH, SQ_SH), :] += rsbuf[1]
    rs2 = rs_step(2, c3)

    contrib(rank)
    rs2.wait_recv()
    out_ref[...] = (partial[pl.ds(rank * SQ_SH, SQ_SH), :]
                    + rsbuf[2]).astype(jnp.float32)

    for d in (d0, d1, d2, rs0, rs1, rs2):
        d.wait_send()

    @functools.partial(pl.run_scoped, exit_sem=pltpu.SemaphoreType.REGULAR)
    def _(exit_sem):
        for nbr in (left, right):
            pl.semaphore_signal(exit_sem, inc=1, device_id=(nbr,),
                                device_id_type=pl.DeviceIdType.MESH)
        pl.semaphore_wait(exit_sem, 2)


def kernel(x, Wq, K_ext, V_ext, Wo):
    rank = lax.axis_index("i")
    bf16 = jnp.bfloat16
    x2 = x.reshape(SQ_SH, D_MODEL).astype(bf16)
    wq3 = Wq.reshape(D_MODEL, H_SH, DH).transpose(1, 0, 2).astype(bf16)
    wo3 = Wo.reshape(H_SH, DH, D_MODEL).astype(bf16)
    ksl = lax.dynamic_slice(
        K_ext[0], (0, rank * H_SH, 0), (SKV_USED, H_SH, DH))
    vsl = lax.dynamic_slice(
        V_ext[0], (0, rank * H_SH, 0), (SKV_USED, H_SH, DH))
    ksl = ksl.astype(bf16).transpose(1, 0, 2)
    vsl = vsl.astype(bf16).transpose(1, 0, 2)

    out = pl.pallas_call(
        _body,
        out_shape=jax.ShapeDtypeStruct((SQ_SH, D_MODEL), jnp.float32),
        in_specs=[
            pl.BlockSpec(memory_space=pltpu.VMEM),
            pl.BlockSpec(memory_space=pltpu.VMEM),
            pl.BlockSpec(memory_space=pl.ANY),
            pl.BlockSpec(memory_space=pl.ANY),
            pl.BlockSpec(memory_space=pltpu.VMEM),
        ],
        out_specs=pl.BlockSpec(memory_space=pltpu.VMEM),
        scratch_shapes=[
            pltpu.VMEM((SQ, D_MODEL), bf16),
            pltpu.VMEM((H_SH, SKV_USED, DH), bf16),
            pltpu.VMEM((H_SH, SKV_USED, DH), bf16),
            pltpu.VMEM((SQ, D_MODEL), bf16),
            pltpu.VMEM((N_DEV - 1, SQ_SH, D_MODEL), bf16),
            pltpu.VMEM((SQ_SH, DH), jnp.float32),
            pltpu.VMEM((SQ_SH, 1), jnp.float32),
            pltpu.SemaphoreType.DMA((N_DEV - 1,)),
            pltpu.SemaphoreType.DMA((N_DEV - 1,)),
            pltpu.SemaphoreType.DMA((N_DEV - 1,)),
            pltpu.SemaphoreType.DMA((N_DEV - 1,)),
            pltpu.SemaphoreType.DMA((2,)),
        ],
        compiler_params=pltpu.CompilerParams(
            collective_id=0,
            vmem_limit_bytes=100 * 1024 * 1024,
        ),
    )(x2, wq3, ksl, vsl, wo3)

    return out.reshape(1, SQ_SH, D_MODEL)
